# baseline (speedup 1.0000x reference)
import functools

import jax
import jax.numpy as jnp
from jax import lax
from jax.experimental import pallas as pl
from jax.experimental.pallas import tpu as pltpu
from jax.experimental.pallas import tpu_sc as plsc

NUM_BEAMS = 8
VOCAB = 51289
CUR_LEN = 32
NW = 32
CHUNK = 12832
VPAD = 4 * CHUNK
NVREG = CHUNK // 16
PRE = 3216
K = 8
NCH = 3
CAND = NCH * K * 16
NEG = -1e30
BIGI = 2**30


def _sc_body(x_hbm, cand_v_hbm, cand_i_hbm, xbuf, vvmem, ivmem, sem1, sem2):
    wid = lax.axis_index("s") * 2 + lax.axis_index("c")
    start = pl.multiple_of(wid * CHUNK, 8)
    c1 = pltpu.make_async_copy(x_hbm.at[pl.ds(start, PRE)],
                               xbuf.at[pl.ds(0, PRE)], sem1)
    c2 = pltpu.make_async_copy(x_hbm.at[pl.ds(start + PRE, CHUNK - PRE)],
                               xbuf.at[pl.ds(PRE, CHUNK - PRE)], sem2)
    c1.start()
    c2.start()
    iota = lax.iota(jnp.int32, 16)
    negv = jnp.full((16,), NEG, jnp.float32)
    bigv = jnp.full((16,), BIGI, jnp.int32)

    def ladder(v, iv, regs):
        out = []
        for j in range(K):
            r, ridx = regs[j], regs[K + j]
            take = v > r
            out.append((jnp.where(take, v, r), jnp.where(take, iv, ridx)))
            v = jnp.where(take, r, v)
            iv = jnp.where(take, ridx, iv)
        return tuple(o[0] for o in out) + tuple(o[1] for o in out)

    def insert(i, carry):
        out = ()
        for t in range(NCH):
            off = i * (NCH * 16) + t * 16
            out = out + ladder(xbuf[pl.ds(off, 16)], iota + off,
                               carry[2 * K * t:2 * K * (t + 1)])
        return out

    init = ((negv,) * K + (bigv,) * K) * NCH
    c1.wait()
    carry = lax.fori_loop(0, PRE // (NCH * 16), insert, init)
    c2.wait()
    carry = lax.fori_loop(PRE // (NCH * 16), NVREG // NCH, insert, carry)
    carry = tuple(carry)
    for t in range((NVREG // NCH) * NCH, NVREG):
        carry = ladder(xbuf[pl.ds(t * 16, 16)], iota + t * 16,
                       carry[:2 * K]) + carry[2 * K:]
    for t in range(NCH):
        for j in range(K):
            vvmem[pl.ds((t * K + j) * 16, 16)] = carry[2 * K * t + j]
            ivmem[pl.ds((t * K + j) * 16, 16)] = carry[2 * K * t + K + j]
    pltpu.sync_copy(vvmem, cand_v_hbm.at[wid])
    pltpu.sync_copy(ivmem, cand_i_hbm.at[wid])


@functools.lru_cache(maxsize=1)
def _sc_scan():
    return pl.kernel(
        _sc_body,
        out_type=[
            jax.ShapeDtypeStruct((NW, CAND), jnp.float32),
            jax.ShapeDtypeStruct((NW, CAND), jnp.int32),
        ],
        mesh=plsc.VectorSubcoreMesh(core_axis_name="c", subcore_axis_name="s"),
        scratch_types=[
            pltpu.VMEM((CHUNK,), jnp.float32),
            pltpu.VMEM((CAND,), jnp.float32),
            pltpu.VMEM((CAND,), jnp.int32),
            pltpu.SemaphoreType.DMA,
            pltpu.SemaphoreType.DMA,
        ],
    )


def _tc_lse(x_ref, lse_ref):
    io = lax.broadcasted_iota(jnp.int32, (1, NUM_BEAMS), 1)
    out = jnp.zeros((1, NUM_BEAMS), jnp.float32)
    for b in range(NUM_BEAMS):
        x = x_ref[pl.ds(b * VPAD, VPAD)]
        mb = jnp.max(x)
        out = jnp.where(io == b, jnp.log(jnp.sum(jnp.exp(x - mb))) + mb, out)
    lse_ref[...] = out


def _tc_merge(cv_ref, ci_ref, lse_ref, bs_ref, dec_ref,
              dec_out_ref, sc_ref, tok_ref, bidx_ref):
    oro = lax.broadcasted_iota(jnp.int32, (NUM_BEAMS, 1), 0)
    lse = jnp.zeros((NUM_BEAMS, 1), jnp.float32)
    for b in range(NUM_BEAMS):
        lse = jnp.where(oro == b, lse_ref[0, b], lse)
    cv = cv_ref[:, :]
    ci = ci_ref[:, :]
    col = lax.broadcasted_iota(jnp.int32, (NUM_BEAMS, 4 * CAND), 1)
    row = lax.broadcasted_iota(jnp.int32, (NUM_BEAMS, 4 * CAND), 0)
    tok = ci + (col // CAND) * CHUNK
    flat = row * VOCAB + tok
    adj = jnp.where(tok < VOCAB, cv - lse + bs_ref[:, :], NEG)

    io8 = lax.broadcasted_iota(jnp.int32, (1, K), 1)
    sc = jnp.zeros((1, K), jnp.float32)
    fl = jnp.zeros((1, K), jnp.int32)
    flats = []
    work = adj
    for j in range(K):
        vmax = jnp.max(work)
        fmin = jnp.min(jnp.where(work == vmax, flat, BIGI))
        work = jnp.where(flat == fmin, NEG, work)
        sc = jnp.where(io8 == j, vmax, sc)
        fl = jnp.where(io8 == j, fmin, fl)
        flats.append(fmin)

    sc_ref[:, :] = sc
    tok_ref[:, :] = fl % VOCAB
    bidx_ref[:, :] = fl // VOCAB

    orow = lax.broadcasted_iota(jnp.int32, (NUM_BEAMS, 1), 0)
    bi_rows = jnp.zeros((NUM_BEAMS, 1), jnp.int32)
    tk_rows = jnp.zeros((NUM_BEAMS, 1), jnp.int32)
    for j in range(K):
        bi_rows = jnp.where(orow == j, flats[j] // VOCAB, bi_rows)
        tk_rows = jnp.where(orow == j, flats[j] % VOCAB, tk_rows)
    reord = jnp.zeros(dec_ref.shape, jnp.int32)
    for k in range(NUM_BEAMS):
        reord = jnp.where(bi_rows == k, dec_ref[k:k + 1, :], reord)
    dec_out_ref[:, 0:CUR_LEN] = reord
    dec_out_ref[:, CUR_LEN:CUR_LEN + 1] = tk_rows


def kernel(lm_logits, beam_scores, decoder_inputs):
    xpad = jnp.pad(lm_logits[:, -1, :], ((0, 0), (0, VPAD - VOCAB)),
                   constant_values=NEG)
    xflat = jnp.reshape(xpad, (-1,))
    cand_v, cand_i = _sc_scan()(xflat)

    lse = pl.pallas_call(
        _tc_lse,
        out_shape=jax.ShapeDtypeStruct((1, NUM_BEAMS), jnp.float32),
    )(xflat)

    cv = jnp.reshape(cand_v, (NUM_BEAMS, 4 * CAND))
    ci = jnp.reshape(cand_i, (NUM_BEAMS, 4 * CAND))
    bs = jnp.reshape(beam_scores, (NUM_BEAMS, 1))

    new_dec, sc, tok, bidx = pl.pallas_call(
        _tc_merge,
        out_shape=[
            jax.ShapeDtypeStruct((NUM_BEAMS, CUR_LEN + 1), jnp.int32),
            jax.ShapeDtypeStruct((1, K), jnp.float32),
            jax.ShapeDtypeStruct((1, K), jnp.int32),
            jax.ShapeDtypeStruct((1, K), jnp.int32),
        ],
    )(cv, ci, lse, bs, decoder_inputs)

    return (new_dec, jnp.reshape(sc, (NUM_BEAMS,)),
            jnp.reshape(tok, (NUM_BEAMS,)), jnp.reshape(bidx, (NUM_BEAMS,)))


# speedup vs baseline: 1.0588x; 1.0588x over previous
import functools

import jax
import jax.numpy as jnp
from jax import lax
from jax.experimental import pallas as pl
from jax.experimental.pallas import tpu as pltpu
from jax.experimental.pallas import tpu_sc as plsc

NUM_BEAMS = 8
VOCAB = 51289
CUR_LEN = 32
NW = 32
CHUNK = 12832
VPAD = 4 * CHUNK
NVREG = CHUNK // 16
PRE = 3200
K = 8
NCH = 2
CAND = NCH * K * 16
NEG = -1e30
BIGI = 2**30


def _sc_body(x_hbm, cand_v_hbm, cand_i_hbm, xbuf, vvmem, ivmem, sem1, sem2):
    wid = lax.axis_index("s") * 2 + lax.axis_index("c")
    start = pl.multiple_of(wid * CHUNK, 8)
    c1 = pltpu.make_async_copy(x_hbm.at[pl.ds(start, PRE)],
                               xbuf.at[pl.ds(0, PRE)], sem1)
    c2 = pltpu.make_async_copy(x_hbm.at[pl.ds(start + PRE, CHUNK - PRE)],
                               xbuf.at[pl.ds(PRE, CHUNK - PRE)], sem2)
    c1.start()
    c2.start()
    iota = lax.iota(jnp.int32, 16)
    negv = jnp.full((16,), NEG, jnp.float32)
    bigv = jnp.full((16,), BIGI, jnp.int32)

    def ladder(v, iv, regs):
        out = []
        for j in range(K):
            r, ridx = regs[j], regs[K + j]
            take = v > r
            out.append((jnp.where(take, v, r), jnp.where(take, iv, ridx)))
            v = jnp.where(take, r, v)
            iv = jnp.where(take, ridx, iv)
        return tuple(o[0] for o in out) + tuple(o[1] for o in out)

    def insert(i, carry):
        out = ()
        for t in range(NCH):
            off = i * (NCH * 16) + t * 16
            out = out + ladder(xbuf[pl.ds(off, 16)], iota + off,
                               carry[2 * K * t:2 * K * (t + 1)])
        return out

    init = ((negv,) * K + (bigv,) * K) * NCH
    c1.wait()
    carry = lax.fori_loop(0, PRE // (NCH * 16), insert, init)
    c2.wait()
    carry = lax.fori_loop(PRE // (NCH * 16), NVREG // NCH, insert, carry)
    carry = tuple(carry)
    for t in range((NVREG // NCH) * NCH, NVREG):
        carry = ladder(xbuf[pl.ds(t * 16, 16)], iota + t * 16,
                       carry[:2 * K]) + carry[2 * K:]
    for t in range(NCH):
        for j in range(K):
            vvmem[pl.ds((t * K + j) * 16, 16)] = carry[2 * K * t + j]
            ivmem[pl.ds((t * K + j) * 16, 16)] = carry[2 * K * t + K + j]
    pltpu.sync_copy(vvmem, cand_v_hbm.at[wid])
    pltpu.sync_copy(ivmem, cand_i_hbm.at[wid])


@functools.lru_cache(maxsize=1)
def _sc_scan():
    return pl.kernel(
        _sc_body,
        out_type=[
            jax.ShapeDtypeStruct((NW, CAND), jnp.float32),
            jax.ShapeDtypeStruct((NW, CAND), jnp.int32),
        ],
        mesh=plsc.VectorSubcoreMesh(core_axis_name="c", subcore_axis_name="s"),
        scratch_types=[
            pltpu.VMEM((CHUNK,), jnp.float32),
            pltpu.VMEM((CAND,), jnp.float32),
            pltpu.VMEM((CAND,), jnp.int32),
            pltpu.SemaphoreType.DMA,
            pltpu.SemaphoreType.DMA,
        ],
    )


def _tc_lse(x_ref, lse_ref):
    io = lax.broadcasted_iota(jnp.int32, (1, NUM_BEAMS), 1)
    out = jnp.zeros((1, NUM_BEAMS), jnp.float32)
    for b in range(NUM_BEAMS):
        x = x_ref[pl.ds(b * VPAD, VPAD)]
        mb = jnp.max(x)
        out = jnp.where(io == b, jnp.log(jnp.sum(jnp.exp(x - mb))) + mb, out)
    lse_ref[...] = out


def _tc_merge(cv_ref, ci_ref, lse_ref, bs_ref, dec_ref,
              dec_out_ref, sc_ref, tok_ref, bidx_ref):
    oro = lax.broadcasted_iota(jnp.int32, (NUM_BEAMS, 1), 0)
    lse = jnp.zeros((NUM_BEAMS, 1), jnp.float32)
    for b in range(NUM_BEAMS):
        lse = jnp.where(oro == b, lse_ref[0, b], lse)
    cv = cv_ref[:, :]
    ci = ci_ref[:, :]
    col = lax.broadcasted_iota(jnp.int32, (NUM_BEAMS, 4 * CAND), 1)
    row = lax.broadcasted_iota(jnp.int32, (NUM_BEAMS, 4 * CAND), 0)
    tok = ci + (col // CAND) * CHUNK
    flat = row * VOCAB + tok
    adj = jnp.where(tok < VOCAB, cv - lse + bs_ref[:, :], NEG)

    io8 = lax.broadcasted_iota(jnp.int32, (1, K), 1)
    sc = jnp.zeros((1, K), jnp.float32)
    fl = jnp.zeros((1, K), jnp.int32)
    flats = []
    work = adj
    for j in range(K):
        vmax = jnp.max(work)
        fmin = jnp.min(jnp.where(work == vmax, flat, BIGI))
        work = jnp.where(flat == fmin, NEG, work)
        sc = jnp.where(io8 == j, vmax, sc)
        fl = jnp.where(io8 == j, fmin, fl)
        flats.append(fmin)

    sc_ref[:, :] = sc
    tok_ref[:, :] = fl % VOCAB
    bidx_ref[:, :] = fl // VOCAB

    orow = lax.broadcasted_iota(jnp.int32, (NUM_BEAMS, 1), 0)
    bi_rows = jnp.zeros((NUM_BEAMS, 1), jnp.int32)
    tk_rows = jnp.zeros((NUM_BEAMS, 1), jnp.int32)
    for j in range(K):
        bi_rows = jnp.where(orow == j, flats[j] // VOCAB, bi_rows)
        tk_rows = jnp.where(orow == j, flats[j] % VOCAB, tk_rows)
    reord = jnp.zeros(dec_ref.shape, jnp.int32)
    for k in range(NUM_BEAMS):
        reord = jnp.where(bi_rows == k, dec_ref[k:k + 1, :], reord)
    dec_out_ref[:, 0:CUR_LEN] = reord
    dec_out_ref[:, CUR_LEN:CUR_LEN + 1] = tk_rows


def kernel(lm_logits, beam_scores, decoder_inputs):
    xpad = jnp.pad(lm_logits[:, -1, :], ((0, 0), (0, VPAD - VOCAB)),
                   constant_values=NEG)
    xflat = jnp.reshape(xpad, (-1,))
    cand_v, cand_i = _sc_scan()(xflat)

    lse = pl.pallas_call(
        _tc_lse,
        out_shape=jax.ShapeDtypeStruct((1, NUM_BEAMS), jnp.float32),
    )(xflat)

    cv = jnp.reshape(cand_v, (NUM_BEAMS, 4 * CAND))
    ci = jnp.reshape(cand_i, (NUM_BEAMS, 4 * CAND))
    bs = jnp.reshape(beam_scores, (NUM_BEAMS, 1))

    new_dec, sc, tok, bidx = pl.pallas_call(
        _tc_merge,
        out_shape=[
            jax.ShapeDtypeStruct((NUM_BEAMS, CUR_LEN + 1), jnp.int32),
            jax.ShapeDtypeStruct((1, K), jnp.float32),
            jax.ShapeDtypeStruct((1, K), jnp.int32),
            jax.ShapeDtypeStruct((1, K), jnp.int32),
        ],
    )(cv, ci, lse, bs, decoder_inputs)

    return (new_dec, jnp.reshape(sc, (NUM_BEAMS,)),
            jnp.reshape(tok, (NUM_BEAMS,)), jnp.reshape(bidx, (NUM_BEAMS,)))
